# baseline (speedup 1.0000x reference)
import functools

import jax
import jax.numpy as jnp
from jax import lax
from jax.experimental import pallas as pl
from jax.experimental.pallas import tpu as pltpu
from jax.experimental.pallas import tpu_sc as plsc

N = 200000
C = 64
NBATCH = 4
NB = 50000
K = 27
G = 32
EPS = 1e-5
CHUNK = 1000
NCH = N // CHUNK
CPB = NB // CHUNK
NTILES = 32
SUB = 128
NPAD = 200704
SPAN = NPAD // NTILES
NSUBCH = SPAN // SUB
NSUB0 = 98
NSUB1 = 0
YROWS = (NCH + 1) * CHUNK
PAIRS = (K + 1) // 2
TW = PAIRS * 128


def _sigmoid(x):
    return 1.0 / (1.0 + jnp.exp(-x))


def _stats_body(x_ref, emb_ref, we_ref, s_ref, ss_ref, eo_ref):
    c = pl.program_id(0)

    @pl.when(c == 0)
    def _():
        e = emb_ref[...]
        se = e * _sigmoid(e)
        eo_ref[...] = jnp.dot(se, we_ref[...], preferred_element_type=jnp.float32)
        s_ref[...] = jnp.zeros_like(s_ref)
        ss_ref[...] = jnp.zeros_like(ss_ref)

    x = x_ref[...]
    b = c // CPB
    cs = jnp.sum(x, axis=0, keepdims=True)
    css = jnp.sum(x * x, axis=0, keepdims=True)
    rows = lax.broadcasted_iota(jnp.int32, (8, C), 0)
    mask = rows == b
    s_ref[...] = s_ref[...] + jnp.where(mask, cs, 0.0)
    ss_ref[...] = ss_ref[...] + jnp.where(mask, css, 0.0)


def _stats_call(x, emb8, we):
    return pl.pallas_call(
        _stats_body,
        grid=(NCH,),
        in_specs=[
            pl.BlockSpec((CHUNK, C), lambda c: (c, 0)),
            pl.BlockSpec((8, 512), lambda c: (0, 0)),
            pl.BlockSpec((512, C), lambda c: (0, 0)),
        ],
        out_specs=[
            pl.BlockSpec((8, C), lambda c: (0, 0)),
            pl.BlockSpec((8, C), lambda c: (0, 0)),
            pl.BlockSpec((8, C), lambda c: (0, 0)),
        ],
        out_shape=[
            jax.ShapeDtypeStruct((8, C), jnp.float32),
            jax.ShapeDtypeStruct((8, C), jnp.float32),
            jax.ShapeDtypeStruct((8, C), jnp.float32),
        ],
    )(x, emb8, we)


def _mm_body(x_ref, scl_ref, sft_ref, w_ref, y_ref):
    c = pl.program_id(0)
    b = jnp.minimum(c // CPB, NBATCH - 1)
    rows = lax.broadcasted_iota(jnp.int32, (8, C), 0)
    sel = rows == b
    scl = jnp.sum(jnp.where(sel, scl_ref[...], 0.0), axis=0, keepdims=True)
    sft = jnp.sum(jnp.where(sel, sft_ref[...], 0.0), axis=0, keepdims=True)
    h = x_ref[...] * scl + sft
    h = h * _sigmoid(h)
    y = jnp.dot(h.astype(jnp.bfloat16), w_ref[...],
                preferred_element_type=jnp.float32)
    y = jnp.where(c >= NCH, 0.0, y)
    for p in range(PAIRS):
        y_ref[p] = y[:, 128 * p:128 * (p + 1)]


def _mm_call(x, scl8, sft8, wcat):
    return pl.pallas_call(
        _mm_body,
        grid=(NCH + 1,),
        in_specs=[
            pl.BlockSpec((CHUNK, C), lambda c: (jnp.minimum(c, NCH - 1), 0)),
            pl.BlockSpec((8, C), lambda c: (0, 0)),
            pl.BlockSpec((8, C), lambda c: (0, 0)),
            pl.BlockSpec((C, TW), lambda c: (0, 0)),
        ],
        out_specs=pl.BlockSpec((PAIRS, CHUNK, 128), lambda c: (0, c, 0)),
        out_shape=jax.ShapeDtypeStruct((PAIRS, YROWS, 128), jnp.float32),
    )(x, scl8, sft8, wcat)


def _sc_conv(tflat, idx3):
    mesh = plsc.VectorSubcoreMesh(core_axis_name="c", subcore_axis_name="s")

    @functools.partial(
        pl.kernel,
        out_type=jax.ShapeDtypeStruct((NPAD, C), jnp.float32),
        mesh=mesh,
        scratch_types=[
            pltpu.VMEM((2, K, SUB), jnp.int32),
            pltpu.VMEM((2, SUB, 128), jnp.float32),
            pltpu.VMEM((2, SUB, 128), jnp.float32),
            pltpu.VMEM((SUB, C), jnp.float32),
            pltpu.SemaphoreType.DMA,
            pltpu.SemaphoreType.DMA,
            pltpu.SemaphoreType.DMA,
            pltpu.SemaphoreType.DMA,
        ],
    )
    def body(t_hbm, idx_hbm, out_hbm, idx_v, acc_a, acc_b, out_v,
             sem_i, sem_g, sem_n, sem_o):
        cc = lax.axis_index("c")
        sid = lax.axis_index("s")
        nsub = jnp.where(cc == 0, NSUB0, NSUB1)
        tch = jnp.where(cc == 0, sid * NSUB0, 16 * NSUB0 + sid * NSUB1)

        def fire_inits(nb, npp):
            pltpu.async_copy(t_hbm.at[nb.at[0]], acc_a.at[npp], sem_n)
            pltpu.async_copy(t_hbm.at[nb.at[1]], acc_b.at[npp], sem_n)

        @pl.when(nsub > 0)
        def _():
            pltpu.async_copy(idx_hbm.at[tch], idx_v.at[0], sem_i).wait()
            fire_inits(idx_v.at[0], 0)
            pltpu.async_copy(idx_hbm.at[tch + 1], idx_v.at[1], sem_i)

        def chunk(ci, carry):
            base = (tch + ci) * SUB
            pp = ci % 2
            ib = idx_v.at[pp]
            aa = acc_a.at[pp]
            ab = acc_b.at[pp]
            pltpu.make_async_copy(t_hbm.at[ib.at[0]], aa, sem_n).wait()
            pltpu.make_async_copy(t_hbm.at[ib.at[1]], ab, sem_n).wait()

            cps = []
            for kk in range(2, K):
                dst = aa if kk % 2 == 0 else ab
                cps.append(
                    pltpu.async_copy(t_hbm.at[ib.at[kk]], dst, sem_g,
                                     add=True))
            for cp in cps:
                cp.wait()

            @pl.when(ci + 1 < nsub)
            def _():
                pltpu.make_async_copy(idx_hbm.at[tch + ci + 1],
                                      idx_v.at[(ci + 1) % 2], sem_i).wait()
                fire_inits(idx_v.at[(ci + 1) % 2], (ci + 1) % 2)

                @pl.when(ci + 2 < nsub)
                def _():
                    pltpu.async_copy(idx_hbm.at[tch + ci + 2],
                                     idx_v.at[pp], sem_i)

            @pl.when(ci >= 1)
            def _():
                pltpu.make_async_copy(out_v, out_hbm.at[pl.ds(base, SUB)],
                                      sem_o).wait()

            def fix(t, carry2):
                r = t // 4
                cc = pl.multiple_of((t % 4) * 16, 16)
                out_v[r, pl.ds(cc, 16)] = (
                    aa[r, pl.ds(cc, 16)] + ab[r, pl.ds(64 + cc, 16)])
                return carry2

            lax.fori_loop(0, SUB * 4, fix, 0)
            pltpu.async_copy(out_v, out_hbm.at[pl.ds(base, SUB)], sem_o)
            return carry

        lax.fori_loop(0, nsub, chunk, 0)

        @pl.when(nsub > 0)
        def _():
            pltpu.make_async_copy(out_v, out_hbm.at[pl.ds(0, SUB)],
                                  sem_o).wait()

    return body(tflat, idx3)


def _final_body(f_ref, x_ref, b_ref, o_ref):
    o_ref[...] = f_ref[...] + x_ref[...] + b_ref[0:1, :]


def _final_call(feats, x2, b2c8):
    return pl.pallas_call(
        _final_body,
        grid=(NCH,),
        in_specs=[
            pl.BlockSpec((CHUNK, C), lambda c: (c, 0)),
            pl.BlockSpec((CHUNK, C), lambda c: (c, 0)),
            pl.BlockSpec((8, C), lambda c: (0, 0)),
        ],
        out_specs=pl.BlockSpec((CHUNK, C), lambda c: (c, 0)),
        out_shape=jax.ShapeDtypeStruct((N, C), jnp.float32),
    )(feats, x2, b2c8)


def _affine_from_sums(s8, ss8, gamma, beta):
    s = s8[:NBATCH]
    ss = ss8[:NBATCH]
    denom = jnp.float32(NB * 2)
    sg = s.reshape(NBATCH, G, 2).sum(-1)
    ssg = ss.reshape(NBATCH, G, 2).sum(-1)
    mean = sg / denom
    var = ssg / denom - mean * mean
    inv = lax.rsqrt(var + EPS)
    invc = jnp.repeat(inv, 2, axis=1)
    meanc = jnp.repeat(mean, 2, axis=1)
    scl = gamma[None, :] * invc
    sft = beta[None, :] - meanc * scl
    return scl, sft


def _pad8(x):
    return jnp.pad(x, ((0, 8 - x.shape[0]), (0, 0)))


def kernel(feats, emb, gamma1, beta1, W1, b1c, We, be, gamma2, beta2, W2,
           b2c, batch_idx, nbrs):
    emb8 = _pad8(emb)
    wc1 = jnp.pad(W1.transpose(1, 0, 2).reshape(C, K * C),
                  ((0, 0), (0, TW - K * C))).astype(jnp.bfloat16)
    wc2 = jnp.pad(W2.transpose(1, 0, 2).reshape(C, K * C),
                  ((0, 0), (0, TW - K * C))).astype(jnp.bfloat16)
    pairbase = (jnp.arange(K, dtype=jnp.int32) // 2 * YROWS)[:, None]
    col = jnp.arange(N, dtype=jnp.int32) % CHUNK
    safe = jnp.where(nbrs == N, N + col[None, :], nbrs)
    idxa = safe + pairbase
    idxa = jnp.pad(idxa, ((0, 0), (0, NPAD - N)))
    idx3 = idxa.reshape(K, NPAD // SUB, SUB).transpose(1, 0, 2)
    b2c8 = jnp.broadcast_to(b2c[None, :], (8, C))

    s8, ss8, eo8 = _stats_call(feats, emb8, We)
    scl1, sft1 = _affine_from_sums(s8, ss8, gamma1, beta1)

    y1 = _mm_call(feats, _pad8(scl1), _pad8(sft1), wc1)
    x1 = _sc_conv(y1.reshape(PAIRS * YROWS, 128), idx3)

    s8b, ss8b, _ = _stats_call(x1, emb8, We)
    d = eo8[:NBATCH] + be[None, :] + b1c[None, :]
    s2 = s8b[:NBATCH] + NB * d
    ss2 = ss8b[:NBATCH] + 2.0 * d * s8b[:NBATCH] + NB * d * d
    scl2, sft2b = _affine_from_sums(_pad8(s2), _pad8(ss2), gamma2, beta2)
    sft2 = d * scl2 + sft2b

    y2 = _mm_call(x1, _pad8(scl2), _pad8(sft2), wc2)
    x2 = _sc_conv(y2.reshape(PAIRS * YROWS, 128), idx3)

    return _final_call(feats, x2, b2c8)


# speedup vs baseline: 1.3273x; 1.3273x over previous
import functools

import jax
import jax.numpy as jnp
from jax import lax
from jax.experimental import pallas as pl
from jax.experimental.pallas import tpu as pltpu
from jax.experimental.pallas import tpu_sc as plsc

N = 200000
C = 64
NBATCH = 4
NB = 50000
K = 27
G = 32
EPS = 1e-5
CHUNK = 1000
NCH = N // CHUNK
CPB = NB // CHUNK
NTILES = 32
SUB = 128
NPAD = 200704
SPAN = NPAD // NTILES
NSUBCH = SPAN // SUB
NSUB0 = 92
NSUB1 = 6
YROWS = (NCH + 1) * CHUNK
PAIRS = (K + 1) // 2
TW = PAIRS * 128


def _sigmoid(x):
    return 1.0 / (1.0 + jnp.exp(-x))


def _stats_body(x_ref, emb_ref, we_ref, s_ref, ss_ref, eo_ref):
    c = pl.program_id(0)

    @pl.when(c == 0)
    def _():
        e = emb_ref[...]
        se = e * _sigmoid(e)
        eo_ref[...] = jnp.dot(se, we_ref[...], preferred_element_type=jnp.float32)
        s_ref[...] = jnp.zeros_like(s_ref)
        ss_ref[...] = jnp.zeros_like(ss_ref)

    x = x_ref[...]
    b = c // CPB
    cs = jnp.sum(x, axis=0, keepdims=True)
    css = jnp.sum(x * x, axis=0, keepdims=True)
    rows = lax.broadcasted_iota(jnp.int32, (8, C), 0)
    mask = rows == b
    s_ref[...] = s_ref[...] + jnp.where(mask, cs, 0.0)
    ss_ref[...] = ss_ref[...] + jnp.where(mask, css, 0.0)


def _stats_call(x, emb8, we):
    return pl.pallas_call(
        _stats_body,
        grid=(NCH,),
        in_specs=[
            pl.BlockSpec((CHUNK, C), lambda c: (c, 0)),
            pl.BlockSpec((8, 512), lambda c: (0, 0)),
            pl.BlockSpec((512, C), lambda c: (0, 0)),
        ],
        out_specs=[
            pl.BlockSpec((8, C), lambda c: (0, 0)),
            pl.BlockSpec((8, C), lambda c: (0, 0)),
            pl.BlockSpec((8, C), lambda c: (0, 0)),
        ],
        out_shape=[
            jax.ShapeDtypeStruct((8, C), jnp.float32),
            jax.ShapeDtypeStruct((8, C), jnp.float32),
            jax.ShapeDtypeStruct((8, C), jnp.float32),
        ],
    )(x, emb8, we)


def _mm_body(x_ref, scl_ref, sft_ref, w_ref, y_ref):
    c = pl.program_id(0)
    b = jnp.minimum(c // CPB, NBATCH - 1)
    rows = lax.broadcasted_iota(jnp.int32, (8, C), 0)
    sel = rows == b
    scl = jnp.sum(jnp.where(sel, scl_ref[...], 0.0), axis=0, keepdims=True)
    sft = jnp.sum(jnp.where(sel, sft_ref[...], 0.0), axis=0, keepdims=True)
    h = x_ref[...] * scl + sft
    h = h * _sigmoid(h)
    y = jnp.dot(h.astype(jnp.bfloat16), w_ref[...],
                preferred_element_type=jnp.float32)
    y = jnp.where(c >= NCH, 0.0, y)
    for p in range(PAIRS):
        y_ref[p] = y[:, 128 * p:128 * (p + 1)]


def _mm_call(x, scl8, sft8, wcat):
    return pl.pallas_call(
        _mm_body,
        grid=(NCH + 1,),
        in_specs=[
            pl.BlockSpec((CHUNK, C), lambda c: (jnp.minimum(c, NCH - 1), 0)),
            pl.BlockSpec((8, C), lambda c: (0, 0)),
            pl.BlockSpec((8, C), lambda c: (0, 0)),
            pl.BlockSpec((C, TW), lambda c: (0, 0)),
        ],
        out_specs=pl.BlockSpec((PAIRS, CHUNK, 128), lambda c: (0, c, 0)),
        out_shape=jax.ShapeDtypeStruct((PAIRS, YROWS, 128), jnp.float32),
    )(x, scl8, sft8, wcat)


def _sc_conv(tflat, idx3):
    mesh = plsc.VectorSubcoreMesh(core_axis_name="c", subcore_axis_name="s")

    @functools.partial(
        pl.kernel,
        out_type=jax.ShapeDtypeStruct((NPAD, C), jnp.float32),
        mesh=mesh,
        scratch_types=[
            pltpu.VMEM((2, K, SUB), jnp.int32),
            pltpu.VMEM((2, SUB, 128), jnp.float32),
            pltpu.VMEM((2, SUB, 128), jnp.float32),
            pltpu.VMEM((SUB, C), jnp.float32),
            pltpu.SemaphoreType.DMA,
            pltpu.SemaphoreType.DMA,
            pltpu.SemaphoreType.DMA,
            pltpu.SemaphoreType.DMA,
        ],
    )
    def body(t_hbm, idx_hbm, out_hbm, idx_v, acc_a, acc_b, out_v,
             sem_i, sem_g, sem_n, sem_o):
        cc = lax.axis_index("c")
        sid = lax.axis_index("s")
        nsub = jnp.where(cc == 0, NSUB0, NSUB1)
        tch = jnp.where(cc == 0, sid * NSUB0, 16 * NSUB0 + sid * NSUB1)

        def fire_inits(nb, npp):
            pltpu.async_copy(t_hbm.at[nb.at[0]], acc_a.at[npp], sem_n)
            pltpu.async_copy(t_hbm.at[nb.at[1]], acc_b.at[npp], sem_n)

        @pl.when(nsub > 0)
        def _():
            pltpu.async_copy(idx_hbm.at[tch], idx_v.at[0], sem_i).wait()
            fire_inits(idx_v.at[0], 0)
            pltpu.async_copy(idx_hbm.at[tch + 1], idx_v.at[1], sem_i)

        def chunk(ci, carry):
            base = (tch + ci) * SUB
            pp = ci % 2
            ib = idx_v.at[pp]
            aa = acc_a.at[pp]
            ab = acc_b.at[pp]
            pltpu.make_async_copy(t_hbm.at[ib.at[0]], aa, sem_n).wait()
            pltpu.make_async_copy(t_hbm.at[ib.at[1]], ab, sem_n).wait()

            cps = []
            for kk in range(2, K):
                dst = aa if kk % 2 == 0 else ab
                cps.append(
                    pltpu.async_copy(t_hbm.at[ib.at[kk]], dst, sem_g,
                                     add=True))
            for cp in cps:
                cp.wait()

            @pl.when(ci + 1 < nsub)
            def _():
                pltpu.make_async_copy(idx_hbm.at[tch + ci + 1],
                                      idx_v.at[(ci + 1) % 2], sem_i).wait()
                fire_inits(idx_v.at[(ci + 1) % 2], (ci + 1) % 2)

                @pl.when(ci + 2 < nsub)
                def _():
                    pltpu.async_copy(idx_hbm.at[tch + ci + 2],
                                     idx_v.at[pp], sem_i)

            @pl.when(ci >= 1)
            def _():
                pltpu.make_async_copy(out_v, out_hbm.at[pl.ds(base, SUB)],
                                      sem_o).wait()

            def fix(t, carry2):
                r = t // 4
                cc = pl.multiple_of((t % 4) * 16, 16)
                out_v[r, pl.ds(cc, 16)] = (
                    aa[r, pl.ds(cc, 16)] + ab[r, pl.ds(64 + cc, 16)])
                return carry2

            lax.fori_loop(0, SUB * 4, fix, 0)
            pltpu.async_copy(out_v, out_hbm.at[pl.ds(base, SUB)], sem_o)
            return carry

        lax.fori_loop(0, nsub, chunk, 0)

        @pl.when(nsub > 0)
        def _():
            pltpu.make_async_copy(out_v, out_hbm.at[pl.ds(0, SUB)],
                                  sem_o).wait()

    return body(tflat, idx3)


def _final_body(f_ref, x_ref, b_ref, o_ref):
    o_ref[...] = f_ref[...] + x_ref[...] + b_ref[0:1, :]


def _final_call(feats, x2, b2c8):
    return pl.pallas_call(
        _final_body,
        grid=(NCH,),
        in_specs=[
            pl.BlockSpec((CHUNK, C), lambda c: (c, 0)),
            pl.BlockSpec((CHUNK, C), lambda c: (c, 0)),
            pl.BlockSpec((8, C), lambda c: (0, 0)),
        ],
        out_specs=pl.BlockSpec((CHUNK, C), lambda c: (c, 0)),
        out_shape=jax.ShapeDtypeStruct((N, C), jnp.float32),
    )(feats, x2, b2c8)


def _affine_from_sums(s8, ss8, gamma, beta):
    s = s8[:NBATCH]
    ss = ss8[:NBATCH]
    denom = jnp.float32(NB * 2)
    sg = s.reshape(NBATCH, G, 2).sum(-1)
    ssg = ss.reshape(NBATCH, G, 2).sum(-1)
    mean = sg / denom
    var = ssg / denom - mean * mean
    inv = lax.rsqrt(var + EPS)
    invc = jnp.repeat(inv, 2, axis=1)
    meanc = jnp.repeat(mean, 2, axis=1)
    scl = gamma[None, :] * invc
    sft = beta[None, :] - meanc * scl
    return scl, sft


def _pad8(x):
    return jnp.pad(x, ((0, 8 - x.shape[0]), (0, 0)))


def kernel(feats, emb, gamma1, beta1, W1, b1c, We, be, gamma2, beta2, W2,
           b2c, batch_idx, nbrs):
    emb8 = _pad8(emb)
    wc1 = jnp.pad(W1.transpose(1, 0, 2).reshape(C, K * C),
                  ((0, 0), (0, TW - K * C))).astype(jnp.bfloat16)
    wc2 = jnp.pad(W2.transpose(1, 0, 2).reshape(C, K * C),
                  ((0, 0), (0, TW - K * C))).astype(jnp.bfloat16)
    pairbase = (jnp.arange(K, dtype=jnp.int32) // 2 * YROWS)[:, None]
    col = jnp.arange(N, dtype=jnp.int32) % CHUNK
    safe = jnp.where(nbrs == N, N + col[None, :], nbrs)
    idxa = safe + pairbase
    idxa = jnp.pad(idxa, ((0, 0), (0, NPAD - N)))
    idx3 = idxa.reshape(K, NPAD // SUB, SUB).transpose(1, 0, 2)
    b2c8 = jnp.broadcast_to(b2c[None, :], (8, C))

    s8, ss8, eo8 = _stats_call(feats, emb8, We)
    scl1, sft1 = _affine_from_sums(s8, ss8, gamma1, beta1)

    y1 = _mm_call(feats, _pad8(scl1), _pad8(sft1), wc1)
    x1 = _sc_conv(y1.reshape(PAIRS * YROWS, 128), idx3)

    s8b, ss8b, _ = _stats_call(x1, emb8, We)
    d = eo8[:NBATCH] + be[None, :] + b1c[None, :]
    s2 = s8b[:NBATCH] + NB * d
    ss2 = ss8b[:NBATCH] + 2.0 * d * s8b[:NBATCH] + NB * d * d
    scl2, sft2b = _affine_from_sums(_pad8(s2), _pad8(ss2), gamma2, beta2)
    sft2 = d * scl2 + sft2b

    y2 = _mm_call(x1, _pad8(scl2), _pad8(sft2), wc2)
    x2 = _sc_conv(y2.reshape(PAIRS * YROWS, 128), idx3)

    return _final_call(feats, x2, b2c8)
